# Initial kernel scaffold; baseline (speedup 1.0000x reference)
import jax
import jax.numpy as jnp
from jax.experimental import pallas as pl


def kernel(tokens, embeddings):
    raise NotImplementedError("write your pallas kernel here")


# speedup vs baseline: 1.1032x; 1.1032x over previous
import functools

import jax
import jax.numpy as jnp
from jax import lax
from jax.experimental import pallas as pl
from jax.experimental.pallas import tpu as pltpu
from jax.experimental.pallas import tpu_sc as plsc

_INFO = plsc.get_sparse_core_info()
_NC = _INFO.num_cores
_NS = _INFO.num_subcores
_NW = _NC * _NS

_CHUNK = 1600


def _gather_sc(flat_idx, table, n, d):
    n_per_w = n // _NW
    n_chunks = n_per_w // _CHUNK
    mesh = plsc.VectorSubcoreMesh(core_axis_name="c", subcore_axis_name="s")

    @functools.partial(
        pl.kernel,
        mesh=mesh,
        out_type=jax.ShapeDtypeStruct((n, d), jnp.float32),
        scratch_types=[
            pltpu.VMEM((_CHUNK,), jnp.int32),
            pltpu.VMEM((_CHUNK, d), jnp.float32),
            pltpu.SemaphoreType.DMA,
        ],
        compiler_params=pltpu.CompilerParams(use_tc_tiling_on_sc=False),
    )
    def k(idx_hbm, table_hbm, out_hbm, idx_v, rows_v, sem):
        wid = lax.axis_index("s") * _NC + lax.axis_index("c")
        base = wid * n_per_w

        def body(i, carry):
            off = base + i * _CHUNK
            pltpu.sync_copy(idx_hbm.at[pl.ds(off, _CHUNK)], idx_v)
            pltpu.async_copy(table_hbm.at[idx_v], rows_v, sem).wait()
            pltpu.sync_copy(rows_v, out_hbm.at[pl.ds(off, _CHUNK)])
            return carry

        lax.fori_loop(0, n_chunks, body, 0)

    return k(flat_idx, table)


def kernel(tokens, embeddings):
    b, h = tokens.shape
    v, d = embeddings.shape
    n = b * h
    flat_idx = tokens.reshape(n).astype(jnp.int32)
    out = _gather_sc(flat_idx, embeddings, n, d)
    return out.reshape(b, h, d)


# speedup vs baseline: 1.1134x; 1.0093x over previous
import functools

import jax
import jax.numpy as jnp
from jax import lax
from jax.experimental import pallas as pl
from jax.experimental.pallas import tpu as pltpu
from jax.experimental.pallas import tpu_sc as plsc

_INFO = plsc.get_sparse_core_info()
_NC = _INFO.num_cores
_NS = _INFO.num_subcores
_NW = _NC * _NS

_CHUNK = 800
_NBUF = 4


def _gather_sc(flat_idx, table, n, d):
    n_per_w = n // _NW
    n_chunks = n_per_w // _CHUNK
    mesh = plsc.VectorSubcoreMesh(core_axis_name="c", subcore_axis_name="s")

    @functools.partial(
        pl.kernel,
        mesh=mesh,
        out_type=jax.ShapeDtypeStruct((n, d), jnp.float32),
        scratch_types=(
            [pltpu.VMEM((_CHUNK,), jnp.int32) for _ in range(_NBUF)]
            + [pltpu.VMEM((_CHUNK, d), jnp.float32) for _ in range(_NBUF)]
            + [pltpu.SemaphoreType.DMA for _ in range(3 * _NBUF)]
        ),
        compiler_params=pltpu.CompilerParams(use_tc_tiling_on_sc=False),
    )
    def k(idx_hbm, table_hbm, out_hbm, *scr):
        idx_bufs = scr[0:_NBUF]
        row_bufs = scr[_NBUF:2 * _NBUF]
        isems = scr[2 * _NBUF:3 * _NBUF]
        gsems = scr[3 * _NBUF:4 * _NBUF]
        osems = scr[4 * _NBUF:5 * _NBUF]

        wid = lax.axis_index("s") * _NC + lax.axis_index("c")
        base = wid * n_per_w

        def idx_start(c, b):
            pltpu.async_copy(
                idx_hbm.at[pl.ds(base + c * _CHUNK, _CHUNK)], idx_bufs[b],
                isems[b])

        def idx_wait(b):
            pltpu.make_async_copy(
                idx_hbm.at[pl.ds(base, _CHUNK)], idx_bufs[b], isems[b]).wait()

        def gather_start(b):
            pltpu.async_copy(table_hbm.at[idx_bufs[b]], row_bufs[b], gsems[b])

        def gather_wait(b):
            pltpu.make_async_copy(
                table_hbm.at[idx_bufs[b]], row_bufs[b], gsems[b]).wait()

        def out_start(c, b):
            pltpu.async_copy(
                row_bufs[b], out_hbm.at[pl.ds(base + c * _CHUNK, _CHUNK)],
                osems[b])

        def out_wait(b):
            pltpu.make_async_copy(
                row_bufs[b], out_hbm.at[pl.ds(base, _CHUNK)], osems[b]).wait()

        for b in range(_NBUF):
            idx_start(b, b)
        idx_wait(0)
        gather_start(0)

        def outer_body(o, carry):
            for b in range(_NBUF):
                g = o * _NBUF + b
                b1 = (b + 1) % _NBUF

                @pl.when(g + 1 < n_chunks)
                def _():
                    @pl.when(g + 1 >= _NBUF)
                    def _():
                        out_wait(b1)
                    idx_wait(b1)
                    gather_start(b1)

                gather_wait(b)
                out_start(g, b)

                @pl.when(g + _NBUF < n_chunks)
                def _():
                    idx_start(g + _NBUF, b)
            return carry

        lax.fori_loop(0, n_chunks // _NBUF, outer_body, 0)
        for b in range(_NBUF):
            out_wait(b)

    return k(flat_idx, table)


def kernel(tokens, embeddings):
    b, h = tokens.shape
    v, d = embeddings.shape
    n = b * h
    flat_idx = tokens.reshape(n).astype(jnp.int32)
    out = _gather_sc(flat_idx, embeddings, n, d)
    return out.reshape(b, h, d)


# speedup vs baseline: 1.7893x; 1.6070x over previous
import functools

import jax
import jax.numpy as jnp
import numpy as np
from jax import lax
from jax.experimental import pallas as pl
from jax.experimental.pallas import tpu as pltpu
from jax.experimental.pallas import tpu_sc as plsc

_INFO = plsc.get_sparse_core_info()
_NC = _INFO.num_cores
_NS = _INFO.num_subcores
_NW = _NC * _NS

_PB = 4
_LANES = 128
_CG = 4


def _gather_sc(idx_flat, table, n_pairs, d):
    pairs_per_w = n_pairs // _NW
    n_stages = pairs_per_w // _PB
    rows_per_stage = _PB * _LANES
    obuf_len = _CG * _PB * 8 * _LANES
    mesh = plsc.VectorSubcoreMesh(core_axis_name="c", subcore_axis_name="s")

    @functools.partial(
        pl.kernel,
        mesh=mesh,
        out_type=jax.ShapeDtypeStruct((n_pairs // _LANES, _CG, _LANES * 8 * _LANES),
                                      jnp.float32),
        scratch_types=(
            [pltpu.VMEM((rows_per_stage,), jnp.int32) for _ in range(2)]
            + [pltpu.VMEM((rows_per_stage, d), jnp.float32) for _ in range(2)]
            + [pltpu.VMEM((obuf_len,), jnp.float32) for _ in range(2)]
            + [pltpu.SemaphoreType.DMA for _ in range(6)]
        ),
        compiler_params=pltpu.CompilerParams(use_tc_tiling_on_sc=False,
                                             needs_layout_passes=False),
    )
    def k(idx_hbm, table_hbm, out_hbm, *scr):
        idx_bufs, row_bufs, obufs = scr[0:2], scr[2:4], scr[4:6]
        isems, gsems, osems = scr[6:8], scr[8:10], scr[10:12]

        wid = lax.axis_index("s") * _NC + lax.axis_index("c")
        p0 = wid * pairs_per_w
        lane = lax.iota(jnp.int32, 16)

        def idx_start(s, b):
            pltpu.async_copy(
                idx_hbm.at[pl.ds((p0 + s * _PB) * _LANES, rows_per_stage)],
                idx_bufs[b], isems[b])

        def idx_wait(b):
            pltpu.make_async_copy(
                idx_hbm.at[pl.ds(0, rows_per_stage)], idx_bufs[b],
                isems[b]).wait()

        def gather_start(b):
            pltpu.async_copy(table_hbm.at[idx_bufs[b]], row_bufs[b], gsems[b])

        def gather_wait(b):
            pltpu.make_async_copy(
                table_hbm.at[idx_bufs[b]], row_bufs[b], gsems[b]).wait()

        def out_start(s, b):
            p = p0 + s * _PB
            h = p // _LANES
            btile = p % _LANES
            for c4 in range(_CG):
                pltpu.async_copy(
                    obufs[b].at[pl.ds(c4 * _PB * 8 * _LANES, _PB * 8 * _LANES)],
                    out_hbm.at[h, c4, pl.ds(btile * 8 * _LANES,
                                            _PB * 8 * _LANES)],
                    osems[b])

        def out_wait(b):
            for c4 in range(_CG):
                pltpu.make_async_copy(
                    obufs[b].at[pl.ds(0, _PB * 8 * _LANES)],
                    out_hbm.at[0, 0, pl.ds(0, _PB * 8 * _LANES)],
                    osems[b]).wait()

        iv_const = []
        for ch in range(d // 16):
            cc = lane + ch * 16
            iv_const.append((cc >> 3) * (_PB * 8 * _LANES)
                            + (cc & 7) * _LANES)

        def transpose(b):
            def rbody(i, carry):
                j2 = i // _LANES
                bl = i % _LANES
                offv = jnp.full((16,), j2 * (8 * _LANES) + bl, jnp.int32)
                for ch in range(d // 16):
                    vec = row_bufs[b][i, pl.ds(ch * 16, 16)]
                    plsc.store_scatter(obufs[b], [iv_const[ch] + offv], vec)
                return carry
            lax.fori_loop(0, rows_per_stage, rbody, 0)

        idx_start(0, 0)
        idx_start(1, 1)
        idx_wait(0)
        gather_start(0)

        def outer_body(o, carry):
            for b in range(2):
                s = o * 2 + b
                nb = 1 - b

                @pl.when(s + 1 < n_stages)
                def _():
                    idx_wait(nb)
                    gather_start(nb)

                gather_wait(b)

                @pl.when(s >= 2)
                def _():
                    out_wait(b)

                transpose(b)
                out_start(s, b)

                @pl.when(s + 2 < n_stages)
                def _():
                    idx_start(s + 2, b)
            return carry

        lax.fori_loop(0, n_stages // 2, outer_body, 0)
        out_wait(0)
        out_wait(1)

    return k(idx_flat, table)


def kernel(tokens, embeddings):
    bsz, hist = tokens.shape
    v, d = embeddings.shape
    n_pairs = hist * (bsz // _LANES)
    idx_flat = jnp.transpose(tokens).reshape(bsz * hist).astype(jnp.int32)
    out_lin = _gather_sc(idx_flat, embeddings, n_pairs, d)
    out = out_lin.reshape(hist, _CG, bsz // _LANES, 8, _LANES)
    out = out.transpose(2, 4, 0, 1, 3).reshape(bsz, hist, d)
    return out
